# baseline (speedup 1.0000x reference)
import jax
import jax.numpy as jnp
from jax.experimental import pallas as pl


def _add_kernel(x_ref, emb_ref, o_ref):
    o_ref[...] = x_ref[...] + emb_ref[...][None, :, :]


def _pos_add_3d(x, emb_slice):
    B, L, D = x.shape
    BLK = 512
    return pl.pallas_call(
        _add_kernel,
        grid=(L // BLK, B),
        in_specs=[
            pl.BlockSpec((1, BLK, D), lambda j, b: (b, j, 0)),
            pl.BlockSpec((BLK, D), lambda j, b: (j, 0)),
        ],
        out_specs=pl.BlockSpec((1, BLK, D), lambda j, b: (b, j, 0)),
        out_shape=jax.ShapeDtypeStruct((B, L, D), x.dtype),
    )(x, emb_slice)


def kernel(x, emb_table):
    if x.ndim == 3:
        L = x.shape[-2]
        return _pos_add_3d(x, emb_table[:L])
    b, h, l, d = x.shape
    xr = jnp.reshape(jnp.transpose(x, (0, 2, 1, 3)), (b, l, h * d))
    xr = _pos_add_3d(xr, emb_table[:l])
    return jnp.transpose(jnp.reshape(xr, (b, l, h, d)), (0, 2, 1, 3))


# speedup vs baseline: 1.0846x; 1.0846x over previous
import jax
import jax.numpy as jnp
from jax.experimental import pallas as pl


def _add_kernel(x_ref, emb_ref, o_ref):
    j = pl.program_id(0)
    BLK = x_ref.shape[1]
    o_ref[...] = x_ref[...] + emb_ref[pl.ds(j * BLK, BLK), :][None, :, :]


def _pos_add_3d(x, emb_slice):
    B, L, D = x.shape
    BLK = 256
    return pl.pallas_call(
        _add_kernel,
        grid=(L // BLK,),
        in_specs=[
            pl.BlockSpec((B, BLK, D), lambda j: (0, j, 0)),
            pl.BlockSpec((L, D), lambda j: (0, 0)),
        ],
        out_specs=pl.BlockSpec((B, BLK, D), lambda j: (0, j, 0)),
        out_shape=jax.ShapeDtypeStruct((B, L, D), x.dtype),
    )(x, emb_slice)


def kernel(x, emb_table):
    if x.ndim == 3:
        L = x.shape[-2]
        return _pos_add_3d(x, emb_table[:L])
    b, h, l, d = x.shape
    xr = jnp.reshape(jnp.transpose(x, (0, 2, 1, 3)), (b, l, h * d))
    xr = _pos_add_3d(xr, emb_table[:l])
    return jnp.transpose(jnp.reshape(xr, (b, l, h, d)), (0, 2, 1, 3))
